# baseline (speedup 1.0000x reference)
import functools

import jax
import jax.numpy as jnp
from jax import lax
from jax.experimental import pallas as pl
from jax.experimental.pallas import tpu as pltpu
from jax.experimental.pallas import tpu_sc as plsc

B, H, W, N, CIN = 2, 191, 191, 20000, 32
C0, C1 = 64, 128
HP, WP = 192, 192
GR = HP * WP
TRASH = GR - 1
CS = 48
NC, NS = 2, 16
NPTS = 20480
NCHUNK, CHUNK = 10, 128
GCHUNK = 5


def _sc_scatter(feats_ext, idx4d, zeros_hbm):
    mesh = plsc.VectorSubcoreMesh(core_axis_name="c", subcore_axis_name="s")
    wb_rows = GR // NS

    @functools.partial(
        pl.kernel, mesh=mesh,
        out_type=jax.ShapeDtypeStruct((NC * GR, CS), jnp.float32),
        scratch_types=[
            pltpu.VMEM((NCHUNK, CHUNK), jnp.int32),
            pltpu.VMEM((CHUNK, CS), jnp.float32),
            pltpu.VMEM_SHARED((GR, CS), jnp.float32),
        ],
    )
    def k(feats_hbm, idx_hbm, z_hbm, out_hbm, idx_v, chunk_v, shared):
        c = lax.axis_index("c")
        s = lax.axis_index("s")
        pltpu.sync_copy(z_hbm.at[pl.ds(s * wb_rows, wb_rows)],
                        shared.at[pl.ds(s * wb_rows, wb_rows)])
        pltpu.sync_copy(idx_hbm.at[c, s], idx_v)
        plsc.subcore_barrier()
        for j in range(NCHUNK):
            pltpu.sync_copy(
                feats_hbm.at[pl.ds((s * NCHUNK + j) * CHUNK, CHUNK)], chunk_v)
        plsc.subcore_barrier()
        pltpu.sync_copy(shared.at[pl.ds(s * wb_rows, wb_rows)],
                        out_hbm.at[pl.ds(c * GR + s * wb_rows, wb_rows)])

    return k(feats_ext, idx4d, zeros_hbm)


def _sc_gather(table, idxg):
    mesh = plsc.VectorSubcoreMesh(core_axis_name="c", subcore_axis_name="s")
    gcs = table.shape[-1]
    rows_per_w = GCHUNK * CHUNK

    @functools.partial(
        pl.kernel, mesh=mesh,
        out_type=jax.ShapeDtypeStruct((NPTS, gcs), jnp.float32),
        scratch_types=[
            pltpu.VMEM((GCHUNK, CHUNK), jnp.int32),
            pltpu.VMEM((rows_per_w, gcs), jnp.float32),
            pltpu.SemaphoreType.DMA,
        ],
    )
    def k(table_hbm, idx_hbm, out_hbm, idx_v, rows_v, sem):
        c = lax.axis_index("c")
        s = lax.axis_index("s")
        wid = s * NC + c
        pltpu.sync_copy(idx_hbm.at[wid], idx_v)
        for j in range(GCHUNK):
            pltpu.async_copy(table_hbm.at[idx_v.at[j]],
                             rows_v.at[pl.ds(j * CHUNK, CHUNK)], sem).wait()
        pltpu.sync_copy(rows_v, out_hbm.at[pl.ds(wid * rows_per_w, rows_per_w)])

    return k(table, idxg)


def _conv3x3(x, cnt, w, th, ident=None, proj=None):
    b, hp, wp, cin = x.shape
    cout = w.shape[-1]
    xp = jnp.pad(x, ((0, 0), (1, 1), (1, 1), (0, 0)))
    views = [xp[:, ky:ky + hp] for ky in range(3)]

    spec_v = pl.BlockSpec((1, th, wp + 2, cin), lambda bb, i: (bb, i, 0, 0))
    spec_c = pl.BlockSpec((1, th, wp, 1), lambda bb, i: (bb, i, 0, 0))
    spec_w = pl.BlockSpec((3, 3, cin, cout), lambda bb, i: (0, 0, 0, 0))
    spec_o = pl.BlockSpec((1, th, wp, cout), lambda bb, i: (bb, i, 0, 0))
    in_specs = [spec_v, spec_v, spec_v, spec_c, spec_w]
    args = views + [cnt, w]
    if ident is not None:
        in_specs.append(pl.BlockSpec((1, th, wp, cout),
                                     lambda bb, i: (bb, i, 0, 0)))
        args.append(ident)
    if proj is not None:
        src, wsc = proj
        cs = src.shape[-1]
        in_specs.append(pl.BlockSpec((1, th, wp, cs),
                                     lambda bb, i: (bb, i, 0, 0)))
        in_specs.append(pl.BlockSpec((cs, cout), lambda bb, i: (0, 0)))
        args += [src, wsc]

    def body(t_ref, m_ref, bo_ref, c_ref, w_ref, *rest):
        o_ref = rest[-1]
        acc = jnp.zeros((th * wp, cout), jnp.float32)
        refs = (t_ref, m_ref, bo_ref)
        for ky in range(3):
            for kx in range(3):
                xk = refs[ky][0, :, kx:kx + wp, :].reshape(th * wp, cin)
                acc += jnp.dot(xk, w_ref[ky, kx],
                               preferred_element_type=jnp.float32)
        m = (c_ref[0].reshape(th * wp, 1) > 0).astype(jnp.float32)
        if proj is not None:
            s_ref, wsc_ref = rest[0], rest[1]
            acc += jnp.dot(s_ref[0].reshape(th * wp, -1), wsc_ref[...],
                           preferred_element_type=jnp.float32)
        out = acc * m
        if ident is not None:
            out += rest[0][0].reshape(th * wp, cout)
        o_ref[0] = out.reshape(th, wp, cout)

    return pl.pallas_call(
        body, grid=(b, hp // th), in_specs=in_specs,
        out_specs=spec_o,
        out_shape=jax.ShapeDtypeStruct((b, hp, wp, cout), jnp.float32),
    )(*args)


def _bn_stats(x, cnt, th=32):
    b, hp, wp, c = x.shape
    x2 = x.reshape(b * hp, wp, c)
    c2 = cnt.reshape(b * hp, wp, 1)

    def body(x_ref, c_ref, o_ref):
        @pl.when(pl.program_id(0) == 0)
        def _():
            o_ref[...] = jnp.zeros_like(o_ref)

        xb = x_ref[...].reshape(th * wp, c)
        m = (c_ref[...].reshape(th * wp, 1) > 0).astype(jnp.float32)
        o_ref[0:1, :] += jnp.sum(xb, axis=0, keepdims=True)
        o_ref[1:2, :] += jnp.sum(xb * xb, axis=0, keepdims=True)
        o_ref[2:3, :] += jnp.sum(m)

    return pl.pallas_call(
        body, grid=(b * hp // th,),
        in_specs=[pl.BlockSpec((th, wp, c), lambda i: (i, 0, 0)),
                  pl.BlockSpec((th, wp, 1), lambda i: (i, 0, 0))],
        out_specs=pl.BlockSpec((8, c), lambda i: (0, 0)),
        out_shape=jax.ShapeDtypeStruct((8, c), jnp.float32),
    )(x2, c2)


def _bn_relu(x, cnt, g, bta, th=32, eps=1e-5):
    b, hp, wp, c = x.shape
    st = _bn_stats(x, cnt, th=th)
    n = st[2, 0]
    mean = st[0] / n
    var = st[1] / n - mean * mean
    scale = g * lax.rsqrt(var + eps)
    shift = bta - mean * scale
    ss = jnp.stack([scale, shift] + [jnp.zeros_like(scale)] * 6)

    def body(x_ref, c_ref, ss_ref, o_ref):
        xb = x_ref[...].reshape(th * wp, c)
        m = (c_ref[...].reshape(th * wp, 1) > 0).astype(jnp.float32)
        y = jax.nn.relu(xb * ss_ref[0:1, :] + ss_ref[1:2, :]) * m
        o_ref[...] = y.reshape(th, wp, c)

    out = pl.pallas_call(
        body, grid=(b * hp // th,),
        in_specs=[pl.BlockSpec((th, wp, c), lambda i: (i, 0, 0)),
                  pl.BlockSpec((th, wp, 1), lambda i: (i, 0, 0)),
                  pl.BlockSpec((8, c), lambda i: (0, 0))],
        out_specs=pl.BlockSpec((th, wp, c), lambda i: (i, 0, 0)),
        out_shape=jax.ShapeDtypeStruct((b * hp, wp, c), jnp.float32),
    )(x.reshape(b * hp, wp, c), cnt.reshape(b * hp, wp, 1), ss)
    return out.reshape(b, hp, wp, c)


def _strided_conv(h, cnt, w, th=24):
    hv, cv = [], []
    for dy in range(3):
        for dx in range(3):
            hv.append(h[:, dy:dy + 190:2, dx:dx + 190:2, :])
            cv.append(cnt[:, dy:dy + 190:2, dx:dx + 190:2, :])
    hcat = jnp.pad(jnp.concatenate(hv, -1), ((0, 0), (0, 1), (0, 0), (0, 0)))
    ccat = jnp.pad(jnp.concatenate(cv, -1), ((0, 0), (0, 1), (0, 0), (0, 0)))
    ws = w.reshape(9 * C0, C1)
    hp2, wp2 = 96, 95

    def body(h_ref, c_ref, w_ref, o_ref, m_ref):
        xb = h_ref[0].reshape(th * wp2, 9 * C0)
        cd = jnp.max(c_ref[0], axis=-1, keepdims=True)
        md = (cd.reshape(th * wp2, 1) > 0).astype(jnp.float32)
        acc = jnp.dot(xb, w_ref[...], preferred_element_type=jnp.float32)
        o_ref[0] = (acc * md).reshape(th, wp2, C1)
        m_ref[0] = cd

    hd, cd = pl.pallas_call(
        body, grid=(B, hp2 // th),
        in_specs=[pl.BlockSpec((1, th, wp2, 9 * C0), lambda bb, i: (bb, i, 0, 0)),
                  pl.BlockSpec((1, th, wp2, 9), lambda bb, i: (bb, i, 0, 0)),
                  pl.BlockSpec((9 * C0, C1), lambda bb, i: (0, 0))],
        out_specs=[pl.BlockSpec((1, th, wp2, C1), lambda bb, i: (bb, i, 0, 0)),
                   pl.BlockSpec((1, th, wp2, 1), lambda bb, i: (bb, i, 0, 0))],
        out_shape=[jax.ShapeDtypeStruct((B, hp2, wp2, C1), jnp.float32),
                   jax.ShapeDtypeStruct((B, hp2, wp2, 1), jnp.float32)],
    )(hcat, ccat, ws)
    return hd, cd


def _res(x, cnt, p, pre, th, wsc=None):
    h = _bn_relu(x, cnt, p[pre + '_g1'], p[pre + '_b1'], th=th)
    h = _conv3x3(h, cnt, p[pre + '_wa'], th)
    h = _bn_relu(h, cnt, p[pre + '_g2'], p[pre + '_b2'], th=th)
    if wsc is None:
        return _conv3x3(h, cnt, p[pre + '_wb'], th, ident=x)
    return _conv3x3(h, cnt, p[pre + '_wb'], th,
                    proj=(x, wsc.reshape(wsc.shape[-2], wsc.shape[-1])))


def kernel(feats, coords, params):
    p = params
    bi, xi, yi = coords[:, 0], coords[:, 1], coords[:, 2]

    feats_ext = jnp.zeros((NPTS, CS), jnp.float32)
    feats_ext = feats_ext.at[:N, :CIN].set(feats)
    feats_ext = feats_ext.at[:N, CIN].set(1.0)
    flat = xi * WP + yi
    idx_all = []
    for c in range(NC):
        idx_c = jnp.where(bi == c, flat, TRASH)
        idx_c = jnp.concatenate([idx_c, jnp.full((NPTS - N,), TRASH, jnp.int32)])
        idx_all.append(idx_c)
    idx4d = jnp.stack(idx_all).reshape(NC, NS, NCHUNK, CHUNK).astype(
        jnp.int32)
    zeros_hbm = jnp.zeros((GR, CS), jnp.float32)
    glob = bi * GR + flat
    dense = jnp.zeros((B * GR, CS), jnp.float32).at[glob].add(
        feats_ext[:N]).reshape(B, GR, CS)
    dense = dense.at[:, TRASH].set(0.0)
    dense = dense.reshape(B, HP, WP, CS)
    x = dense[..., :CIN]
    cnt = dense[..., CIN:CIN + 1]

    th = 16
    x0 = _conv3x3(x, cnt, p['w0'], th)
    x1 = _res(x0, cnt, p, 'r0', th)
    h = _bn_relu(x1, cnt, p['g3'], p['b3'], th=th)
    hd, cnt_d = _strided_conv(h, cnt, p['ws'])
    x2 = _res(hd, cnt_d, p, 'r1', 16)
    hb = _bn_relu(x2, cnt_d, p['g6'], p['b6'], th=16)
    hb95 = hb[:, :95, :95]
    xe = lax.pad(hb95, 0.0, ((0, 0, 0), (1, 1, 1), (1, 1, 1), (0, 0, 0)))
    xe = jnp.pad(xe, ((0, 0), (0, 1), (0, 1), (0, 0)))
    up = _conv3x3(xe, cnt, p['wt'], th)
    xc = jnp.concatenate([x1, up], axis=-1)
    x3 = _res(xc, cnt, p, 'r2', th, wsc=p['wsc'])
    y = _bn_relu(x3, cnt, p['g9'], p['b9'], th=th)

    table = jnp.pad(y.reshape(B * GR, C0), ((0, 0), (0, 128 - C0)))
    gflat = bi * GR + xi * WP + yi
    gflat = jnp.concatenate([gflat, jnp.zeros((NPTS - N,), jnp.int32)])
    idxg = gflat.reshape(NC * NS, GCHUNK, CHUNK).astype(jnp.int32)
    if True:
        out = jnp.take(table, idxg.reshape(-1), axis=0)
    else:
        out = _sc_gather(table, idxg)
    return out[:N, :C0]
